# baseline (speedup 1.0000x reference)
import jax
import jax.numpy as jnp
from jax import lax
from jax.experimental import pallas as pl
from jax.experimental.pallas import tpu as pltpu
from jax.experimental.pallas import tpu_sc as plsc

_B = 4096
_L = 200
_G = 5
_D = 64
_NC = 2
_NS = 16
_NW = _NC * _NS
_BPW = _B // _NW


_NACC = 4


def _sc_body(idxt_hbm, itab_hbm, gtab_hbm, uorder_hbm,
             h_out, h2_out, h3_out, h4_out, g_out,
             hidx_v, hacc1_v, hacc2_v, hacc3_v, hacc4_v, gidx_v, gacc_v,
             gsem, hsem1, hsem2, hsem3, hsem4):
    del uorder_hbm
    wid = lax.axis_index("s") * _NC + lax.axis_index("c")
    base = wid * _BPW
    haccs = (hacc1_v, hacc2_v, hacc3_v, hacc4_v)
    hsems = (hsem1, hsem2, hsem3, hsem4)
    houts = (h_out, h2_out, h3_out, h4_out)
    seg = _L // _NACC

    pltpu.sync_copy(idxt_hbm.at[pl.ds(0, _L), pl.ds(base, _BPW)], hidx_v)
    pltpu.sync_copy(idxt_hbm.at[pl.ds(_L, _G), pl.ds(base, _BPW)], gidx_v)

    gcp = pltpu.async_copy(gtab_hbm.at[gidx_v.at[0]], gacc_v, gsem)
    hcps = [pltpu.async_copy(itab_hbm.at[hidx_v.at[a * seg]], haccs[a],
                             hsems[a]) for a in range(_NACC)]

    gcp.wait()
    for g in range(1, _G):
        pltpu.async_copy(gtab_hbm.at[gidx_v.at[g]], gacc_v, gsem, add=True)

    for cp in hcps:
        cp.wait()

    def hfire(l, carry):
        for a in range(_NACC):
            pltpu.async_copy(itab_hbm.at[hidx_v.at[l + a * seg]],
                             haccs[a], hsems[a], add=True)
        return carry

    lax.fori_loop(1, seg, hfire, 0)

    for g in range(1, _G):
        pltpu.make_async_copy(gtab_hbm.at[gidx_v.at[0]], gacc_v, gsem).wait()
    pltpu.sync_copy(gacc_v, g_out.at[pl.ds(base, _BPW)])

    def hdrain(l, carry):
        for a in range(_NACC):
            pltpu.make_async_copy(itab_hbm.at[hidx_v.at[0]], haccs[a],
                                  hsems[a]).wait()
        return carry

    lax.fori_loop(1, seg, hdrain, 0)
    for a in range(_NACC):
        pltpu.sync_copy(haccs[a], houts[a].at[pl.ds(base, _BPW)])


def _sc_user_body(uid_hbm, utab_t_hbm, u_out, uidx_v, tbuf_v, ucol_v,
                  sem0, sem1, sem2, sem3):
    wid = lax.axis_index("s") * _NC + lax.axis_index("c")
    base = wid * _BPW

    pltpu.sync_copy(uid_hbm.at[pl.ds(base, _BPW)], uidx_v)

    iota16 = lax.iota(jnp.int32, 16)
    nbuf = 4
    sems = (sem0, sem1, sem2, sem3)

    def fire(uid, slot):
        col0 = (uid // 128) * 128
        pltpu.async_copy(utab_t_hbm.at[:, pl.ds(col0, 128)],
                         tbuf_v.at[pl.ds(slot * _D, _D), :], sems[slot])

    def wait(slot):
        pltpu.make_async_copy(
            utab_t_hbm.at[:, pl.ds(0, 128)],
            tbuf_v.at[pl.ds(slot * _D, _D), :], sems[slot]).wait()

    def extract(uid, slot, j):
        lane = uid - (uid // 128) * 128
        for r0 in range(0, _D, 16):
            rows = iota16 + (slot * _D + r0)
            vals = plsc.load_gather(
                tbuf_v, [rows, jnp.broadcast_to(lane, (16,))])
            plsc.store_scatter(
                ucol_v, [iota16 + r0, jnp.broadcast_to(j, (16,))], vals)

    def chunk(jc, carry):
        vec = uidx_v[pl.ds(jc * 16, 16)]
        for j in range(nbuf):
            fire(vec[j], j)
        for j in range(16):
            wait(j % nbuf)
            extract(vec[j], j % nbuf, jc * 16 + j)
            if j + nbuf < 16:
                fire(vec[j + nbuf], (j + nbuf) % nbuf)
        return carry

    lax.fori_loop(0, _BPW // 16, chunk, 0)
    pltpu.sync_copy(ucol_v, u_out.at[:, pl.ds(base, _BPW)])


def _sc_user_gather(user_id, user_table_t):
    mesh = plsc.VectorSubcoreMesh(core_axis_name="c", subcore_axis_name="s")
    f = pl.kernel(
        _sc_user_body,
        out_type=jax.ShapeDtypeStruct((_D, _B), jnp.float32),
        mesh=mesh,
        scratch_types=[
            pltpu.VMEM((_BPW,), jnp.int32),
            pltpu.VMEM((4 * _D, 128), jnp.float32),
            pltpu.VMEM((_D, _BPW), jnp.float32),
            pltpu.SemaphoreType.DMA,
            pltpu.SemaphoreType.DMA,
            pltpu.SemaphoreType.DMA,
            pltpu.SemaphoreType.DMA,
        ],
        compiler_params=pltpu.CompilerParams(needs_layout_passes=False),
    )
    return f(user_id, user_table_t)


def _sc_gather(idx_t, item_table, genre_table, u_emb_t):
    mesh = plsc.VectorSubcoreMesh(core_axis_name="c", subcore_axis_name="s")
    f = pl.kernel(
        _sc_body,
        out_type=tuple(jax.ShapeDtypeStruct((_B, _D), jnp.float32)
                       for _ in range(_NACC + 1)),
        mesh=mesh,
        scratch_types=(
            [pltpu.VMEM((_L, _BPW), jnp.int32)]
            + [pltpu.VMEM((_BPW, _D), jnp.float32) for _ in range(_NACC)]
            + [pltpu.VMEM((_G, _BPW), jnp.int32),
               pltpu.VMEM((_BPW, _D), jnp.float32)]
            + [pltpu.SemaphoreType.DMA for _ in range(_NACC + 1)]
        ),
        compiler_params=pltpu.CompilerParams(use_tc_tiling_on_sc=False),
    )
    return f(idx_t, item_table, genre_table, u_emb_t)


_BT = 512


_NIDX = _L + _G + 3


def _prep_body(hist_ref, genre_ref, idxt_ref, hcnt_ref, gcnt_ref):
    hist = hist_ref[...]
    genre = genre_ref[...]
    idxt_ref[...] = jnp.concatenate(
        [hist.T, genre.T, jnp.zeros((3, hist.shape[0]), jnp.int32)], axis=0)
    hcnt_ref[...] = jnp.sum((hist > 0).astype(jnp.float32), axis=1,
                            keepdims=True)
    gcnt_ref[...] = jnp.sum((genre > 0).astype(jnp.float32), axis=1,
                            keepdims=True)


def _tc_prep(history, top_genres, interpret=False):
    grid = (_B // _BT,)
    return pl.pallas_call(
        _prep_body,
        grid=grid,
        in_specs=[
            pl.BlockSpec((_BT, _L), lambda i: (i, 0)),
            pl.BlockSpec((_BT, _G), lambda i: (i, 0)),
        ],
        out_specs=[
            pl.BlockSpec((_NIDX, _BT), lambda i: (0, i)),
            pl.BlockSpec((_BT, 1), lambda i: (i, 0)),
            pl.BlockSpec((_BT, 1), lambda i: (i, 0)),
        ],
        out_shape=[
            jax.ShapeDtypeStruct((_NIDX, _B), jnp.int32),
            jax.ShapeDtypeStruct((_B, 1), jnp.float32),
            jax.ShapeDtypeStruct((_B, 1), jnp.float32),
        ],
        interpret=interpret,
    )(history, top_genres)


def _tc_body(hcnt_ref, gcnt_ref, cf_ref, ut_ref, hs1_ref, hs2_ref,
             hs3_ref, hs4_ref, gs_ref,
             wc_ref, bc_ref, w1_ref, b1_ref, w2_ref, b2_ref, out_ref):
    hsum = ((hs1_ref[...] + hs2_ref[...]) + (hs3_ref[...] + hs4_ref[...]))
    h = hsum / (hcnt_ref[...] + 1e-8)
    g = gs_ref[...] / (gcnt_ref[...] + 1e-8)
    cont = jnp.maximum(
        jnp.dot(cf_ref[...], wc_ref[...], preferred_element_type=jnp.float32)
        + bc_ref[...], 0.0)
    x = jnp.concatenate([ut_ref[...].T, h, g, cont], axis=1)
    h1 = jnp.maximum(
        jnp.dot(x, w1_ref[...], preferred_element_type=jnp.float32)
        + b1_ref[...], 0.0)
    o = jnp.dot(h1, w2_ref[...], preferred_element_type=jnp.float32) + b2_ref[...]
    norm = jnp.sqrt(jnp.sum(o * o, axis=1, keepdims=True))
    out_ref[...] = (o / jnp.maximum(norm, 1e-12)).T


def _tc_mlp(hcnt, gcnt, cf, u_emb_t, h_sums, g_sum,
            W_cont, b_cont, W1, b1, W2, b2, interpret=False):
    grid = (_B // _BT,)
    row = lambda i: (i, 0)
    rep = lambda i: (0, 0)
    return pl.pallas_call(
        _tc_body,
        grid=grid,
        in_specs=[
            pl.BlockSpec((_BT, 1), row),
            pl.BlockSpec((_BT, 1), row),
            pl.BlockSpec((_BT, 2), row),
            pl.BlockSpec((_D, _BT), lambda i: (0, i)),
        ] + [pl.BlockSpec((_BT, _D), row) for _ in range(_NACC)] + [
            pl.BlockSpec((_BT, _D), row),
            pl.BlockSpec((2, _D), rep),
            pl.BlockSpec((1, _D), rep),
            pl.BlockSpec((4 * _D, 128), rep),
            pl.BlockSpec((1, 128), rep),
            pl.BlockSpec((128, _D), rep),
            pl.BlockSpec((1, _D), rep),
        ],
        out_specs=pl.BlockSpec((_D, _BT), lambda i: (0, i)),
        out_shape=jax.ShapeDtypeStruct((_D, _B), jnp.float32),
        interpret=interpret,
    )(hcnt, gcnt, cf, u_emb_t, *h_sums, g_sum,
      W_cont, b_cont.reshape(1, _D), W1, b1.reshape(1, 128),
      W2, b2.reshape(1, _D))


def kernel(user_id, history, top_genres, avg_rating, activity,
           user_table, item_table, genre_table,
           W_cont, b_cont, W1, b1, W2, b2):
    u_emb_t = _sc_user_gather(user_id, user_table.T)
    idx_t, hcnt, gcnt = _tc_prep(history, top_genres)
    *h_sums, g_sum = _sc_gather(idx_t, item_table, genre_table, u_emb_t)
    cf = jnp.stack([avg_rating, activity], axis=1)
    return _tc_mlp(hcnt, gcnt, cf, u_emb_t, h_sums, g_sum,
                   W_cont, b_cont, W1, b1, W2, b2).T


# speedup vs baseline: 1.0106x; 1.0106x over previous
import jax
import jax.numpy as jnp
from jax import lax
from jax.experimental import pallas as pl
from jax.experimental.pallas import tpu as pltpu
from jax.experimental.pallas import tpu_sc as plsc

_B = 4096
_L = 200
_G = 5
_D = 64
_NC = 2
_NS = 16
_NW = _NC * _NS
_BPW = _B // _NW


_NACC = 2


def _sc_body(idxt_hbm, itab_hbm, gtab_hbm, uorder_hbm,
             h_out, h2_out, g_out,
             hidx_v, hacc1_v, hacc2_v, gidx_v, gacc_v,
             gsem, hsem1, hsem2):
    del uorder_hbm
    wid = lax.axis_index("s") * _NC + lax.axis_index("c")
    base = wid * _BPW
    haccs = (hacc1_v, hacc2_v)
    hsems = (hsem1, hsem2)
    houts = (h_out, h2_out)
    seg = _L // _NACC

    pltpu.sync_copy(idxt_hbm.at[pl.ds(0, _L), pl.ds(base, _BPW)], hidx_v)
    pltpu.sync_copy(idxt_hbm.at[pl.ds(_L, _G), pl.ds(base, _BPW)], gidx_v)

    gcp = pltpu.async_copy(gtab_hbm.at[gidx_v.at[0]], gacc_v, gsem)
    hcps = [pltpu.async_copy(itab_hbm.at[hidx_v.at[a * seg]], haccs[a],
                             hsems[a]) for a in range(_NACC)]

    gcp.wait()
    for g in range(1, _G):
        pltpu.async_copy(gtab_hbm.at[gidx_v.at[g]], gacc_v, gsem, add=True)

    for cp in hcps:
        cp.wait()

    def hfire(l, carry):
        for a in range(_NACC):
            pltpu.async_copy(itab_hbm.at[hidx_v.at[l + a * seg]],
                             haccs[a], hsems[a], add=True)
        return carry

    lax.fori_loop(1, seg, hfire, 0)

    for g in range(1, _G):
        pltpu.make_async_copy(gtab_hbm.at[gidx_v.at[0]], gacc_v, gsem).wait()
    pltpu.sync_copy(gacc_v, g_out.at[pl.ds(base, _BPW)])

    def hdrain(l, carry):
        for a in range(_NACC):
            pltpu.make_async_copy(itab_hbm.at[hidx_v.at[0]], haccs[a],
                                  hsems[a]).wait()
        return carry

    lax.fori_loop(1, seg, hdrain, 0)
    for a in range(_NACC):
        pltpu.sync_copy(haccs[a], houts[a].at[pl.ds(base, _BPW)])


def _sc_user_body(uid_hbm, utab_t_hbm, u_out, uidx_v, tbuf_v, ucol_v,
                  sem0, sem1, sem2, sem3):
    wid = lax.axis_index("s") * _NC + lax.axis_index("c")
    base = wid * _BPW

    pltpu.sync_copy(uid_hbm.at[pl.ds(base, _BPW)], uidx_v)

    iota16 = lax.iota(jnp.int32, 16)
    nbuf = 4
    sems = (sem0, sem1, sem2, sem3)

    def fire(uid, slot):
        col0 = (uid // 128) * 128
        pltpu.async_copy(utab_t_hbm.at[:, pl.ds(col0, 128)],
                         tbuf_v.at[pl.ds(slot * _D, _D), :], sems[slot])

    def wait(slot):
        pltpu.make_async_copy(
            utab_t_hbm.at[:, pl.ds(0, 128)],
            tbuf_v.at[pl.ds(slot * _D, _D), :], sems[slot]).wait()

    def extract(uid, slot, j):
        lane = uid - (uid // 128) * 128
        for r0 in range(0, _D, 16):
            rows = iota16 + (slot * _D + r0)
            vals = plsc.load_gather(
                tbuf_v, [rows, jnp.broadcast_to(lane, (16,))])
            plsc.store_scatter(
                ucol_v, [iota16 + r0, jnp.broadcast_to(j, (16,))], vals)

    def chunk(jc, carry):
        vec = uidx_v[pl.ds(jc * 16, 16)]
        for j in range(nbuf):
            fire(vec[j], j)
        for j in range(16):
            wait(j % nbuf)
            extract(vec[j], j % nbuf, jc * 16 + j)
            if j + nbuf < 16:
                fire(vec[j + nbuf], (j + nbuf) % nbuf)
        return carry

    lax.fori_loop(0, _BPW // 16, chunk, 0)
    pltpu.sync_copy(ucol_v, u_out.at[:, pl.ds(base, _BPW)])


def _sc_user_gather(user_id, user_table_t):
    mesh = plsc.VectorSubcoreMesh(core_axis_name="c", subcore_axis_name="s")
    f = pl.kernel(
        _sc_user_body,
        out_type=jax.ShapeDtypeStruct((_D, _B), jnp.float32),
        mesh=mesh,
        scratch_types=[
            pltpu.VMEM((_BPW,), jnp.int32),
            pltpu.VMEM((4 * _D, 128), jnp.float32),
            pltpu.VMEM((_D, _BPW), jnp.float32),
            pltpu.SemaphoreType.DMA,
            pltpu.SemaphoreType.DMA,
            pltpu.SemaphoreType.DMA,
            pltpu.SemaphoreType.DMA,
        ],
        compiler_params=pltpu.CompilerParams(needs_layout_passes=False),
    )
    return f(user_id, user_table_t)


def _sc_gather(idx_t, item_table, genre_table, u_emb_t):
    mesh = plsc.VectorSubcoreMesh(core_axis_name="c", subcore_axis_name="s")
    f = pl.kernel(
        _sc_body,
        out_type=tuple(jax.ShapeDtypeStruct((_B, _D), jnp.float32)
                       for _ in range(_NACC + 1)),
        mesh=mesh,
        scratch_types=(
            [pltpu.VMEM((_L, _BPW), jnp.int32)]
            + [pltpu.VMEM((_BPW, _D), jnp.float32) for _ in range(_NACC)]
            + [pltpu.VMEM((_G, _BPW), jnp.int32),
               pltpu.VMEM((_BPW, _D), jnp.float32)]
            + [pltpu.SemaphoreType.DMA for _ in range(_NACC + 1)]
        ),
        compiler_params=pltpu.CompilerParams(use_tc_tiling_on_sc=False),
    )
    return f(idx_t, item_table, genre_table, u_emb_t)


_BT = 512


_NIDX = _L + _G + 3


def _prep_body(hist_ref, genre_ref, idxt_ref, hcnt_ref, gcnt_ref):
    hist = hist_ref[...]
    genre = genre_ref[...]
    idxt_ref[...] = jnp.concatenate(
        [hist.T, genre.T, jnp.zeros((3, hist.shape[0]), jnp.int32)], axis=0)
    hcnt_ref[...] = jnp.sum((hist > 0).astype(jnp.float32), axis=1,
                            keepdims=True)
    gcnt_ref[...] = jnp.sum((genre > 0).astype(jnp.float32), axis=1,
                            keepdims=True)


def _tc_prep(history, top_genres, interpret=False):
    grid = (_B // _BT,)
    return pl.pallas_call(
        _prep_body,
        grid=grid,
        in_specs=[
            pl.BlockSpec((_BT, _L), lambda i: (i, 0)),
            pl.BlockSpec((_BT, _G), lambda i: (i, 0)),
        ],
        out_specs=[
            pl.BlockSpec((_NIDX, _BT), lambda i: (0, i)),
            pl.BlockSpec((_BT, 1), lambda i: (i, 0)),
            pl.BlockSpec((_BT, 1), lambda i: (i, 0)),
        ],
        out_shape=[
            jax.ShapeDtypeStruct((_NIDX, _B), jnp.int32),
            jax.ShapeDtypeStruct((_B, 1), jnp.float32),
            jax.ShapeDtypeStruct((_B, 1), jnp.float32),
        ],
        interpret=interpret,
    )(history, top_genres)


def _tc_body(hcnt_ref, gcnt_ref, cf_ref, ut_ref, hs1_ref, hs2_ref,
             gs_ref,
             wc_ref, bc_ref, w1_ref, b1_ref, w2_ref, b2_ref, out_ref):
    hsum = hs1_ref[...] + hs2_ref[...]
    h = hsum / (hcnt_ref[...] + 1e-8)
    g = gs_ref[...] / (gcnt_ref[...] + 1e-8)
    cont = jnp.maximum(
        jnp.dot(cf_ref[...], wc_ref[...], preferred_element_type=jnp.float32)
        + bc_ref[...], 0.0)
    x = jnp.concatenate([ut_ref[...].T, h, g, cont], axis=1)
    h1 = jnp.maximum(
        jnp.dot(x, w1_ref[...], preferred_element_type=jnp.float32)
        + b1_ref[...], 0.0)
    o = jnp.dot(h1, w2_ref[...], preferred_element_type=jnp.float32) + b2_ref[...]
    norm = jnp.sqrt(jnp.sum(o * o, axis=1, keepdims=True))
    out_ref[...] = (o / jnp.maximum(norm, 1e-12)).T


def _tc_mlp(hcnt, gcnt, cf, u_emb_t, h_sums, g_sum,
            W_cont, b_cont, W1, b1, W2, b2, interpret=False):
    grid = (_B // _BT,)
    row = lambda i: (i, 0)
    rep = lambda i: (0, 0)
    return pl.pallas_call(
        _tc_body,
        grid=grid,
        in_specs=[
            pl.BlockSpec((_BT, 1), row),
            pl.BlockSpec((_BT, 1), row),
            pl.BlockSpec((_BT, 2), row),
            pl.BlockSpec((_D, _BT), lambda i: (0, i)),
        ] + [pl.BlockSpec((_BT, _D), row) for _ in range(_NACC)] + [
            pl.BlockSpec((_BT, _D), row),
            pl.BlockSpec((2, _D), rep),
            pl.BlockSpec((1, _D), rep),
            pl.BlockSpec((4 * _D, 128), rep),
            pl.BlockSpec((1, 128), rep),
            pl.BlockSpec((128, _D), rep),
            pl.BlockSpec((1, _D), rep),
        ],
        out_specs=pl.BlockSpec((_D, _BT), lambda i: (0, i)),
        out_shape=jax.ShapeDtypeStruct((_D, _B), jnp.float32),
        interpret=interpret,
    )(hcnt, gcnt, cf, u_emb_t, *h_sums, g_sum,
      W_cont, b_cont.reshape(1, _D), W1, b1.reshape(1, 128),
      W2, b2.reshape(1, _D))


def kernel(user_id, history, top_genres, avg_rating, activity,
           user_table, item_table, genre_table,
           W_cont, b_cont, W1, b1, W2, b2):
    u_emb_t = _sc_user_gather(user_id, user_table.T)
    idx_t, hcnt, gcnt = _tc_prep(history, top_genres)
    *h_sums, g_sum = _sc_gather(idx_t, item_table, genre_table, u_emb_t)
    cf = jnp.stack([avg_rating, activity], axis=1)
    return _tc_mlp(hcnt, gcnt, cf, u_emb_t, h_sums, g_sum,
                   W_cont, b_cont, W1, b1, W2, b2).T
